# baseline (speedup 1.0000x reference)
import functools

import jax
import jax.numpy as jnp
from jax.experimental import pallas as pl
from jax.experimental.pallas import tpu as pltpu


def _gru_body(mem_ref, agg_ref, cnt_ref, wih_ref, whh_ref, bih_ref, bhh_ref,
              out_ref):
    mem = mem_ref[...]
    agg = agg_ref[...]
    gi = jnp.dot(agg, wih_ref[...], preferred_element_type=jnp.float32)
    gi = gi + bih_ref[...][None, :]
    gh = jnp.dot(mem, whh_ref[...], preferred_element_type=jnp.float32)
    gh = gh + bhh_ref[...][None, :]
    D = mem.shape[1]
    i_r, i_z, i_n = gi[:, :D], gi[:, D:2 * D], gi[:, 2 * D:]
    h_r, h_z, h_n = gh[:, :D], gh[:, D:2 * D], gh[:, 2 * D:]
    r = jax.nn.sigmoid(i_r + h_r)
    z = jax.nn.sigmoid(i_z + h_z)
    n = jnp.tanh(i_n + r * h_n)
    new_h = (1.0 - z) * n + z * mem
    out_ref[...] = jnp.where(cnt_ref[...] > 0, new_h, mem)


def kernel(mem, idx, val, t, W_ih, W_hh, b_ih, b_hh):
    Mn, Dn = mem.shape
    ones = jnp.ones_like(t)
    counts = jax.ops.segment_sum(ones, idx, num_segments=Mn)
    t_max = jax.ops.segment_max(t, idx, num_segments=Mn)
    w = jnp.exp(t - t_max[idx])
    msg_sum = jax.ops.segment_sum(val * w[:, None], idx, num_segments=Mn)
    w_sum = jax.ops.segment_sum(w, idx, num_segments=Mn)
    agg = msg_sum / jnp.maximum(w_sum, 1e-6)[:, None]

    BLK = 2000
    grid = Mn // BLK
    out = pl.pallas_call(
        _gru_body,
        grid=(grid,),
        in_specs=[
            pl.BlockSpec((BLK, Dn), lambda i: (i, 0)),
            pl.BlockSpec((BLK, Dn), lambda i: (i, 0)),
            pl.BlockSpec((BLK, 1), lambda i: (i, 0)),
            pl.BlockSpec((Dn, 3 * Dn), lambda i: (0, 0)),
            pl.BlockSpec((Dn, 3 * Dn), lambda i: (0, 0)),
            pl.BlockSpec((3 * Dn,), lambda i: (0,)),
            pl.BlockSpec((3 * Dn,), lambda i: (0,)),
        ],
        out_specs=pl.BlockSpec((BLK, Dn), lambda i: (i, 0)),
        out_shape=jax.ShapeDtypeStruct((Mn, Dn), jnp.float32),
    )(mem, agg, counts[:, None], W_ih, W_hh, b_ih, b_hh)
    return out
